# baseline (speedup 1.0000x reference)
import functools

import jax
import jax.numpy as jnp
from jax import lax
from jax.experimental import pallas as pl
from jax.experimental.pallas import tpu as pltpu
from jax.experimental.pallas import tpu_sc as plsc

N = 10000
DEG = 32
D = 128
OUT_D = 64
BETA = 0.2

NC, NS = 2, 16
NW = NC * NS
CH = 2
N_CH = N // CH
E = N_CH * DEG
E_PER_W = E // NW
GB = 40
NB = E_PER_W // GB
SLOTS = 5

B = 200


def _sc_gather(x, col3):
    mesh = plsc.VectorSubcoreMesh(core_axis_name="c", subcore_axis_name="s",
                                  num_cores=NC, num_subcores=NS)

    def body(x_hbm, col_hbm, out_hbm, idx_v, rows_v, *sems):
        gsems, osems = sems[:SLOTS], sems[SLOTS:]
        wid = lax.axis_index("s") * NC + lax.axis_index("c")
        pltpu.sync_copy(col_hbm.at[wid], idx_v)
        row0 = lambda b: (b * NW + wid) * GB

        for s in range(SLOTS):
            pltpu.async_copy(x_hbm.at[idx_v.at[s]], rows_v.at[s], gsems[s])

        def step(i, carry):
            for s in range(SLOTS):
                b = i * SLOTS + s
                pltpu.make_async_copy(x_hbm.at[idx_v.at[b]], rows_v.at[s],
                                      gsems[s]).wait()
                pltpu.async_copy(rows_v.at[s],
                                 out_hbm.at[pl.ds(row0(b), GB)],
                                 osems[s])

                @pl.when(b + SLOTS < NB)
                def _():
                    pltpu.make_async_copy(
                        rows_v.at[s], out_hbm.at[pl.ds(row0(b), GB)],
                        osems[s]).wait()
                    pltpu.async_copy(x_hbm.at[idx_v.at[b + SLOTS]],
                                     rows_v.at[s], gsems[s])
            return carry

        lax.fori_loop(0, NB // SLOTS, step, 0)
        for s in range(SLOTS):
            pltpu.make_async_copy(
                rows_v.at[s],
                out_hbm.at[pl.ds(row0(NB - SLOTS + s), GB)],
                osems[s]).wait()

    return pl.kernel(
        body,
        out_type=jax.ShapeDtypeStruct((E, D), jnp.float32),
        mesh=mesh,
        scratch_types=[
            pltpu.VMEM((NB, GB), jnp.int32),
            pltpu.VMEM((SLOTS, GB, D), jnp.float32),
        ] + [pltpu.SemaphoreType.DMA] * (2 * SLOTS),
    )(x, col3)


def _tc_body(x_ref, g_ref,
             wn1_ref, bn1_ref, ws1_ref, bs1_ref, wa1_ref, ba1_ref, wb1_ref, bb1_ref,
             wn2_ref, bn2_ref, ws2_ref, bs2_ref, wa2_ref, ba2_ref, wb2_ref, bb2_ref,
             o_ref, dwt_ref):
    xb = x_ref[...]
    g = g_ref[...]
    gflat = g.reshape(DEG * B, D)
    diff = xb[None, :, :] - g
    dwt_ref[...] = jnp.sum(diff * diff, axis=-1)
    nd = jnp.sqrt(dwt_ref[...]) * (1.0 / (250.0 + 1e-8))
    dwt = (jnp.exp(-0.3 * nd) * 1.4426950408889634)[:, :, None]

    def attn(tgt, wn_ref, bn_ref, ws_ref, bs_ref, wa_ref, ba_ref, wb_ref, bb_ref):
        sg = jnp.dot(gflat, wn_ref[...], preferred_element_type=jnp.float32) + bn_ref[...]
        e = jnp.exp2(sg.reshape(DEG, B, D) * dwt)
        denom = jnp.sum(e, axis=0)
        nbout = jnp.sum(e * g, axis=0) / denom
        sa = jnp.dot(tgt, ws_ref[...], preferred_element_type=jnp.float32) + bs_ref[...]
        ctx = BETA * (sa * tgt) + (1.0 - BETA) * nbout
        h = jnp.dot(ctx, wa_ref[...], preferred_element_type=jnp.float32) + ba_ref[...]
        h = jnp.maximum(h, 0.01 * h)
        o = jnp.dot(h, wb_ref[...], preferred_element_type=jnp.float32) + bb_ref[...]
        return jnp.maximum(o, 0.0)

    y1 = attn(xb, wn1_ref, bn1_ref, ws1_ref, bs1_ref, wa1_ref, ba1_ref, wb1_ref, bb1_ref)
    o_ref[...] = attn(y1, wn2_ref, bn2_ref, ws2_ref, bs2_ref, wa2_ref, ba2_ref, wb2_ref, bb2_ref)


def _tc_fused(c, x, G2, *weights):
    wspecs = [pl.BlockSpec(w.shape, lambda i: (0, 0)) for w in weights]
    off = c * (N_CH // B)
    return pl.pallas_call(
        _tc_body,
        grid=(N_CH // B,),
        in_specs=[
            pl.BlockSpec((B, D), lambda i: (i + off, 0)),
            pl.BlockSpec((DEG, B, D), lambda i: (0, i, 0)),
        ] + wspecs,
        out_specs=pl.BlockSpec((B, OUT_D), lambda i: (i, 0)),
        out_shape=jax.ShapeDtypeStruct((N_CH, OUT_D), jnp.float32),
        scratch_shapes=[pltpu.VMEM((DEG, B), jnp.float32)],
        compiler_params=pltpu.CompilerParams(
            dimension_semantics=("arbitrary",),
        ),
    )(x, G2, *weights)


def kernel(x, edge_index, W_self1, b_self1, W_nb1, b_nb1, W_ffn1a, b_ffn1a,
           W_ffn1b, b_ffn1b, W_self2, b_self2, W_nb2, b_nb2, W_ffn2a, b_ffn2a,
           W_ffn2b, b_ffn2b):
    colT = edge_index[1].astype(jnp.int32).reshape(N, DEG).T
    r = lambda b: b.reshape(1, -1)
    w = (W_nb1.T, r(b_nb1), W_self1.T, r(b_self1), W_ffn1a.T, r(b_ffn1a), W_ffn1b.T, r(b_ffn1b),
         W_nb2.T, r(b_nb2), W_self2.T, r(b_self2), W_ffn2a.T, r(b_ffn2a), W_ffn2b.T, r(b_ffn2b))
    gs = []
    for c in range(CH):
        cc = colT[:, c * N_CH:(c + 1) * N_CH].reshape(NB, NW, GB).transpose(1, 0, 2)
        gs.append(_sc_gather(x, cc).reshape(DEG, N_CH, D))
    outs = [_tc_fused(c, x, gs[c], *w) for c in range(CH)]
    return jnp.concatenate(outs, axis=0) if CH > 1 else outs[0]


# speedup vs baseline: 1.0450x; 1.0450x over previous
import jax
import jax.numpy as jnp
from jax import lax
from jax.experimental import pallas as pl
from jax.experimental.pallas import tpu as pltpu
from jax.experimental.pallas import tpu_sc as plsc

N = 10000
DEG = 32
D = 128
OUT_D = 64
BETA = 0.2

NC, NS = 2, 16
NW = NC * NS
CH = 5
N_CH = N // CH
E = N_CH * DEG
E_PER_W = E // NW
GB = 80
NB = E_PER_W // GB
SLOTS = 5

B = 400


def _sc_gather(x, col3):
    mesh = plsc.VectorSubcoreMesh(core_axis_name="c", subcore_axis_name="s",
                                  num_cores=NC, num_subcores=NS)

    def body(x_hbm, col_hbm, out_hbm, idx_v, rows_v, *sems):
        gsems, osems = sems[:SLOTS], sems[SLOTS:]
        wid = lax.axis_index("s") * NC + lax.axis_index("c")
        pltpu.sync_copy(col_hbm.at[wid], idx_v)
        row0 = lambda b: (b * NW + wid) * GB

        for s in range(SLOTS):
            pltpu.async_copy(x_hbm.at[idx_v.at[s]], rows_v.at[s], gsems[s])

        def step(i, carry):
            for s in range(SLOTS):
                b = i * SLOTS + s
                pltpu.make_async_copy(x_hbm.at[idx_v.at[b]], rows_v.at[s],
                                      gsems[s]).wait()
                pltpu.async_copy(rows_v.at[s],
                                 out_hbm.at[pl.ds(row0(b), GB)],
                                 osems[s])

                @pl.when(b + SLOTS < NB)
                def _():
                    pltpu.make_async_copy(
                        rows_v.at[s], out_hbm.at[pl.ds(row0(b), GB)],
                        osems[s]).wait()
                    pltpu.async_copy(x_hbm.at[idx_v.at[b + SLOTS]],
                                     rows_v.at[s], gsems[s])
            return carry

        lax.fori_loop(0, NB // SLOTS, step, 0)
        for s in range(SLOTS):
            pltpu.make_async_copy(
                rows_v.at[s],
                out_hbm.at[pl.ds(row0(NB - SLOTS + s), GB)],
                osems[s]).wait()

    return pl.kernel(
        body,
        out_type=jax.ShapeDtypeStruct((E, D), jnp.float32),
        mesh=mesh,
        scratch_types=[
            pltpu.VMEM((NB, GB), jnp.int32),
            pltpu.VMEM((SLOTS, GB, D), jnp.float32),
        ] + [pltpu.SemaphoreType.DMA] * (2 * SLOTS),
    )(x, col3)


def _tc_body(x_ref, g_ref,
             wn1_ref, bn1_ref, ws1_ref, bs1_ref, wa1_ref, ba1_ref, wb1_ref, bb1_ref,
             wn2_ref, bn2_ref, ws2_ref, bs2_ref, wa2_ref, ba2_ref, wb2_ref, bb2_ref,
             o_ref, dwt_ref):
    xb = x_ref[...]
    g = g_ref[...]
    gflat = g.reshape(DEG * B, D)
    diff = xb[None, :, :] - g
    dwt_ref[...] = jnp.sum(diff * diff, axis=-1)
    nd = jnp.sqrt(dwt_ref[...]) * (1.0 / (250.0 + 1e-8))
    dwt = (jnp.exp(-0.3 * nd) * 1.4426950408889634)[:, :, None]

    def attn(tgt, wn_ref, bn_ref, ws_ref, bs_ref, wa_ref, ba_ref, wb_ref, bb_ref):
        sg = jnp.dot(gflat, wn_ref[...], preferred_element_type=jnp.float32) + bn_ref[...]
        e = jnp.exp2(sg.reshape(DEG, B, D) * dwt)
        denom = jnp.sum(e, axis=0)
        nbout = jnp.sum(e * g, axis=0) / denom
        sa = jnp.dot(tgt, ws_ref[...], preferred_element_type=jnp.float32) + bs_ref[...]
        ctx = BETA * (sa * tgt) + (1.0 - BETA) * nbout
        h = jnp.dot(ctx, wa_ref[...], preferred_element_type=jnp.float32) + ba_ref[...]
        h = jnp.maximum(h, 0.01 * h)
        o = jnp.dot(h, wb_ref[...], preferred_element_type=jnp.float32) + bb_ref[...]
        return jnp.maximum(o, 0.0)

    y1 = attn(xb, wn1_ref, bn1_ref, ws1_ref, bs1_ref, wa1_ref, ba1_ref, wb1_ref, bb1_ref)
    o_ref[...] = attn(y1, wn2_ref, bn2_ref, ws2_ref, bs2_ref, wa2_ref, ba2_ref, wb2_ref, bb2_ref)


def _tc_fused(c, x, G2, *weights):
    wspecs = [pl.BlockSpec(w.shape, lambda i: (0, 0)) for w in weights]
    off = c * (N_CH // B)
    return pl.pallas_call(
        _tc_body,
        grid=(N_CH // B,),
        in_specs=[
            pl.BlockSpec((B, D), lambda i: (i + off, 0)),
            pl.BlockSpec((DEG, B, D), lambda i: (0, i, 0)),
        ] + wspecs,
        out_specs=pl.BlockSpec((B, OUT_D), lambda i: (i, 0)),
        out_shape=jax.ShapeDtypeStruct((N_CH, OUT_D), jnp.float32),
        scratch_shapes=[pltpu.VMEM((DEG, B), jnp.float32)],
        compiler_params=pltpu.CompilerParams(
            dimension_semantics=("arbitrary",),
        ),
    )(x, G2, *weights)


def kernel(x, edge_index, W_self1, b_self1, W_nb1, b_nb1, W_ffn1a, b_ffn1a,
           W_ffn1b, b_ffn1b, W_self2, b_self2, W_nb2, b_nb2, W_ffn2a, b_ffn2a,
           W_ffn2b, b_ffn2b):
    colT = edge_index[1].astype(jnp.int32).reshape(N, DEG).T
    r = lambda b: b.reshape(1, -1)
    w = (W_nb1.T, r(b_nb1), W_self1.T, r(b_self1), W_ffn1a.T, r(b_ffn1a), W_ffn1b.T, r(b_ffn1b),
         W_nb2.T, r(b_nb2), W_self2.T, r(b_self2), W_ffn2a.T, r(b_ffn2a), W_ffn2b.T, r(b_ffn2b))
    gs = []
    for c in range(CH):
        cc = colT[:, c * N_CH:(c + 1) * N_CH].reshape(NB, NW, GB).transpose(1, 0, 2)
        gs.append(_sc_gather(x, cc).reshape(DEG, N_CH, D))
    outs = [_tc_fused(c, x, gs[c], *w) for c in range(CH)]
    return jnp.concatenate(outs, axis=0) if CH > 1 else outs[0]


# speedup vs baseline: 1.0465x; 1.0015x over previous
import jax
import jax.numpy as jnp
from jax import lax
from jax.experimental import pallas as pl
from jax.experimental.pallas import tpu as pltpu
from jax.experimental.pallas import tpu_sc as plsc

N = 10000
DEG = 32
D = 128
OUT_D = 64
BETA = 0.2

NC, NS = 2, 16
NW = NC * NS
CH = 5
N_CH = N // CH
E = N_CH * DEG
E_PER_W = E // NW
GB = 80
NB = E_PER_W // GB
SLOTS = 5

B = 400


def _sc_gather(x, col3):
    mesh = plsc.VectorSubcoreMesh(core_axis_name="c", subcore_axis_name="s",
                                  num_cores=NC, num_subcores=NS)

    def body(x_hbm, col_hbm, out_hbm, idx_v, rows_v, *sems):
        gsems, osems = sems[:SLOTS], sems[SLOTS:]
        wid = lax.axis_index("s") * NC + lax.axis_index("c")
        pltpu.sync_copy(col_hbm.at[wid], idx_v)
        row0 = lambda b: (b * NW + wid) * GB

        for s in range(SLOTS):
            pltpu.async_copy(x_hbm.at[idx_v.at[s]], rows_v.at[s], gsems[s])

        def step(i, carry):
            for s in range(SLOTS):
                b = i * SLOTS + s
                pltpu.make_async_copy(x_hbm.at[idx_v.at[b]], rows_v.at[s],
                                      gsems[s]).wait()
                pltpu.async_copy(rows_v.at[s],
                                 out_hbm.at[pl.ds(row0(b), GB)],
                                 osems[s])

                @pl.when(b + SLOTS < NB)
                def _():
                    pltpu.make_async_copy(
                        rows_v.at[s], out_hbm.at[pl.ds(row0(b), GB)],
                        osems[s]).wait()
                    pltpu.async_copy(x_hbm.at[idx_v.at[b + SLOTS]],
                                     rows_v.at[s], gsems[s])
            return carry

        lax.fori_loop(0, NB // SLOTS, step, 0)
        for s in range(SLOTS):
            pltpu.make_async_copy(
                rows_v.at[s],
                out_hbm.at[pl.ds(row0(NB - SLOTS + s), GB)],
                osems[s]).wait()

    return pl.kernel(
        body,
        out_type=jax.ShapeDtypeStruct((E, D), jnp.float32),
        mesh=mesh,
        scratch_types=[
            pltpu.VMEM((NB, GB), jnp.int32),
            pltpu.VMEM((SLOTS, GB, D), jnp.float32),
        ] + [pltpu.SemaphoreType.DMA] * (2 * SLOTS),
    )(x, col3)


def _tc_body(x_ref, g_ref,
             wn1_ref, bn1_ref, ws1_ref, bs1_ref, wa1_ref, ba1_ref, wb1_ref, bb1_ref,
             wn2_ref, bn2_ref, ws2_ref, bs2_ref, wa2_ref, ba2_ref, wb2_ref, bb2_ref,
             o_ref, dwt_ref):
    xb = x_ref[...]
    g = g_ref[...]
    gflat = g.reshape(DEG * B, D)
    diff = xb[None, :, :] - g
    dwt_ref[...] = jnp.sum(diff * diff, axis=-1)
    nd = jnp.sqrt(dwt_ref[...]) * (1.0 / (250.0 + 1e-8))
    dwt = (jnp.exp(-0.3 * nd) * 1.4426950408889634)[:, :, None]

    def attn(tgt, wn_ref, bn_ref, ws_ref, bs_ref, wa_ref, ba_ref, wb_ref, bb_ref):
        sg = jnp.dot(gflat, wn_ref[...], preferred_element_type=jnp.float32) + bn_ref[...]
        e = jnp.exp2(sg.reshape(DEG, B, D) * dwt)
        denom = jnp.sum(e, axis=0)
        nbout = jnp.sum(e * g, axis=0) / denom
        sa = jnp.dot(tgt, ws_ref[...], preferred_element_type=jnp.float32) + bs_ref[...]
        ctx = BETA * (sa * tgt) + (1.0 - BETA) * nbout
        h = jnp.dot(ctx, wa_ref[...], preferred_element_type=jnp.float32) + ba_ref[...]
        h = jnp.maximum(h, 0.01 * h)
        o = jnp.dot(h, wb_ref[...], preferred_element_type=jnp.float32) + bb_ref[...]
        return jnp.maximum(o, 0.0)

    y1 = attn(xb, wn1_ref, bn1_ref, ws1_ref, bs1_ref, wa1_ref, ba1_ref, wb1_ref, bb1_ref)
    o_ref[...] = attn(y1, wn2_ref, bn2_ref, ws2_ref, bs2_ref, wa2_ref, ba2_ref, wb2_ref, bb2_ref)


def _tc_fused(c, x, G2, *weights):
    wspecs = [pl.BlockSpec(w.shape, lambda i: (0, 0)) for w in weights]
    off = c * (N_CH // B)
    return pl.pallas_call(
        _tc_body,
        grid=(N_CH // B,),
        in_specs=[
            pl.BlockSpec((B, D), lambda i: (i + off, 0)),
            pl.BlockSpec((DEG, B, D), lambda i: (0, i, 0)),
        ] + wspecs,
        out_specs=pl.BlockSpec((B, OUT_D), lambda i: (i, 0)),
        out_shape=jax.ShapeDtypeStruct((N_CH, OUT_D), jnp.float32),
        scratch_shapes=[pltpu.VMEM((DEG, B), jnp.float32)],
        compiler_params=pltpu.CompilerParams(
            dimension_semantics=("arbitrary",),
        ),
    )(x, G2, *weights)


def kernel(x, edge_index, W_self1, b_self1, W_nb1, b_nb1, W_ffn1a, b_ffn1a,
           W_ffn1b, b_ffn1b, W_self2, b_self2, W_nb2, b_nb2, W_ffn2a, b_ffn2a,
           W_ffn2b, b_ffn2b):
    colT = edge_index[1].astype(jnp.int32).reshape(N, DEG).T
    r = lambda b: b.reshape(1, -1)
    w = (W_nb1.T, r(b_nb1), W_self1.T, r(b_self1), W_ffn1a.T, r(b_ffn1a), W_ffn1b.T, r(b_ffn1b),
         W_nb2.T, r(b_nb2), W_self2.T, r(b_self2), W_ffn2a.T, r(b_ffn2a), W_ffn2b.T, r(b_ffn2b))
    col4 = (colT.reshape(DEG, CH, N_CH).transpose(1, 0, 2)
            .reshape(CH, NB, NW, GB).transpose(0, 2, 1, 3))
    gs = []
    for c in range(CH):
        gs.append(_sc_gather(x, col4[c]).reshape(DEG, N_CH, D))
    outs = [_tc_fused(c, x, gs[c], *w) for c in range(CH)]
    return jnp.concatenate(outs, axis=0) if CH > 1 else outs[0]


# speedup vs baseline: 1.0475x; 1.0009x over previous
import jax
import jax.numpy as jnp
from jax import lax
from jax.experimental import pallas as pl
from jax.experimental.pallas import tpu as pltpu
from jax.experimental.pallas import tpu_sc as plsc

N = 10000
DEG = 32
D = 128
OUT_D = 64
BETA = 0.2

NC, NS = 2, 16
NW = NC * NS
CH = 5
N_CH = N // CH
E = N_CH * DEG
E_PER_W = E // NW
GB = 80
NB = E_PER_W // GB
SLOTS = 5

B = 400


def _sc_gather(x, col3):
    mesh = plsc.VectorSubcoreMesh(core_axis_name="c", subcore_axis_name="s",
                                  num_cores=NC, num_subcores=NS)

    def body(x_hbm, col_hbm, out_hbm, idx_v, rows_v, *sems):
        gsems, osems = sems[:SLOTS], sems[SLOTS:]
        wid = lax.axis_index("s") * NC + lax.axis_index("c")
        pltpu.sync_copy(col_hbm.at[wid], idx_v)
        row0 = lambda b: (b * NW + wid) * GB

        for s in range(SLOTS):
            pltpu.async_copy(x_hbm.at[idx_v.at[s]], rows_v.at[s], gsems[s])

        def step(i, carry):
            for s in range(SLOTS):
                b = i * SLOTS + s
                pltpu.make_async_copy(x_hbm.at[idx_v.at[b]], rows_v.at[s],
                                      gsems[s]).wait()
                pltpu.async_copy(rows_v.at[s],
                                 out_hbm.at[pl.ds(row0(b), GB)],
                                 osems[s])

                @pl.when(b + SLOTS < NB)
                def _():
                    pltpu.make_async_copy(
                        rows_v.at[s], out_hbm.at[pl.ds(row0(b), GB)],
                        osems[s]).wait()
                    pltpu.async_copy(x_hbm.at[idx_v.at[b + SLOTS]],
                                     rows_v.at[s], gsems[s])
            return carry

        lax.fori_loop(0, NB // SLOTS, step, 0)
        for s in range(SLOTS):
            pltpu.make_async_copy(
                rows_v.at[s],
                out_hbm.at[pl.ds(row0(NB - SLOTS + s), GB)],
                osems[s]).wait()

    return pl.kernel(
        body,
        out_type=jax.ShapeDtypeStruct((E, D), jnp.float32),
        mesh=mesh,
        scratch_types=[
            pltpu.VMEM((NB, GB), jnp.int32),
            pltpu.VMEM((SLOTS, GB, D), jnp.float32),
        ] + [pltpu.SemaphoreType.DMA] * (2 * SLOTS),
    )(x, col3)


def _tc_body(x_ref, g_ref,
             wn1_ref, bn1_ref, ws1_ref, bs1_ref, wa1_ref, ba1_ref, wb1_ref, bb1_ref,
             wn2_ref, bn2_ref, ws2_ref, bs2_ref, wa2_ref, ba2_ref, wb2_ref, bb2_ref,
             o_ref, dwt_ref):
    xb = x_ref[...]
    g = g_ref[...]
    gflat = g.reshape(DEG * B, D)
    diff = xb[None, :, :] - g
    dwt_ref[...] = jnp.sum(diff * diff, axis=-1)
    nd = jnp.sqrt(dwt_ref[...]) * (1.0 / (250.0 + 1e-8))
    dwt = (jnp.exp(-0.3 * nd) * 1.4426950408889634)[:, :, None]

    def attn(tgt, wn_ref, bn_ref, ws_ref, bs_ref, wa_ref, ba_ref, wb_ref, bb_ref):
        sg = jnp.dot(gflat, wn_ref[...], preferred_element_type=jnp.float32) + bn_ref[...]
        e = jnp.exp2(sg.reshape(DEG, B, D) * dwt)
        denom = jnp.sum(e, axis=0)
        nbout = jnp.sum(e * g, axis=0) / denom
        sa = jnp.dot(tgt, ws_ref[...], preferred_element_type=jnp.float32) + bs_ref[...]
        ctx = BETA * (sa * tgt) + (1.0 - BETA) * nbout
        h = jnp.dot(ctx, wa_ref[...], preferred_element_type=jnp.float32) + ba_ref[...]
        h = jnp.maximum(h, 0.01 * h)
        o = jnp.dot(h, wb_ref[...], preferred_element_type=jnp.float32) + bb_ref[...]
        return jnp.maximum(o, 0.0)

    y1 = attn(xb, wn1_ref, bn1_ref, ws1_ref, bs1_ref, wa1_ref, ba1_ref, wb1_ref, bb1_ref)
    o_ref[...] = attn(y1, wn2_ref, bn2_ref, ws2_ref, bs2_ref, wa2_ref, ba2_ref, wb2_ref, bb2_ref)


def _tc_fused(c, x, G2, *weights):
    wspecs = [pl.BlockSpec(w.shape, lambda i: (0, 0)) for w in weights]
    off = c * (N_CH // B)
    return pl.pallas_call(
        _tc_body,
        grid=(N_CH // B,),
        in_specs=[
            pl.BlockSpec((B, D), lambda i: (i + off, 0)),
            pl.BlockSpec((DEG, B, D), lambda i: (0, i, 0)),
        ] + wspecs,
        out_specs=pl.BlockSpec((B, OUT_D), lambda i: (i, 0)),
        out_shape=jax.ShapeDtypeStruct((N_CH, OUT_D), jnp.float32),
        scratch_shapes=[pltpu.VMEM((DEG, B), jnp.float32)],
        compiler_params=pltpu.CompilerParams(
            dimension_semantics=("arbitrary",),
        ),
    )(x, G2, *weights)


def kernel(x, edge_index, W_self1, b_self1, W_nb1, b_nb1, W_ffn1a, b_ffn1a,
           W_ffn1b, b_ffn1b, W_self2, b_self2, W_nb2, b_nb2, W_ffn2a, b_ffn2a,
           W_ffn2b, b_ffn2b):
    colT = edge_index[1].astype(jnp.int32).reshape(N, DEG).T
    r = lambda b: b.reshape(1, -1)
    w = (W_nb1.T, r(b_nb1), W_self1.T, r(b_self1), W_ffn1a.T, r(b_ffn1a), W_ffn1b.T, r(b_ffn1b),
         W_nb2.T, r(b_nb2), W_self2.T, r(b_self2), W_ffn2a.T, r(b_ffn2a), W_ffn2b.T, r(b_ffn2b))
    col4 = (colT.reshape(DEG, CH, N_CH).transpose(1, 0, 2)
            .reshape(CH, NB, NW, GB).transpose(0, 2, 1, 3))
    gs, outs = [], []
    gs.append(_sc_gather(x, col4[0]).reshape(DEG, N_CH, D))
    for c in range(1, CH):
        gs.append(_sc_gather(x, col4[c]).reshape(DEG, N_CH, D))
        outs.append(_tc_fused(c - 1, x, gs[c - 1], *w))
    outs.append(_tc_fused(CH - 1, x, gs[CH - 1], *w))
    return jnp.concatenate(outs, axis=0) if CH > 1 else outs[0]
